# baseline (speedup 1.0000x reference)
import functools

import jax
import jax.numpy as jnp
import numpy as np
from jax import lax
from jax.experimental import pallas as pl
from jax.experimental.pallas import tpu as pltpu
from jax.experimental.pallas import tpu_sc as plsc

ALPHA = 0.5
BETA = 1.0
M_EDGES = 80000


def _sc_two_hop(N, E, M, D, *, L=32, DT=jnp.bfloat16, NC=2, NS=16, BLK=125,
                K=1, RING=4, ZR=125, ZRF=50, interpret=False):
    HL = L // 2
    S = D // L
    SPC = S // NC
    SB = K * BLK
    NSB = E // (NS * SB)
    RPT = E // (NS * BLK)
    STRIPE_M = M // NS
    STRIPE_N = N // NS
    ZCOP = STRIPE_M // ZR
    NZCOP = STRIPE_N // ZRF
    NZREM = STRIPE_N - NZCOP * ZRF
    assert S * L == D and SPC * NC == S
    assert NS * SB * NSB == E and RPT == NSB * K
    assert ZCOP * ZR == STRIPE_M and STRIPE_N * NS == N
    assert BLK <= 128 and RING == 4 and NSB >= 4

    mesh = plsc.VectorSubcoreMesh(core_axis_name="core", subcore_axis_name="sub",
                                  num_cores=NC, num_subcores=NS)

    @functools.partial(
        pl.kernel,
        out_type=jax.ShapeDtypeStruct((N, D), jnp.float32),
        mesh=mesh,
        interpret=interpret,
        compiler_params=pltpu.CompilerParams(use_tc_tiling_on_sc=False,
                                             needs_layout_passes=False),
        scratch_types=[
            pltpu.VMEM_SHARED((M, L), DT),
            pltpu.VMEM_SHARED((N, L), DT),
            pltpu.VMEM_SHARED((N, L), jnp.float32),
            pltpu.VMEM((RING, K, BLK), jnp.int32),
            pltpu.VMEM((RING, K, BLK), jnp.int32),
            pltpu.VMEM((RING, K, BLK, L), DT),
            pltpu.VMEM((2, K, BLK, L), jnp.float32),
            pltpu.VMEM((ZR, L), DT),
            pltpu.VMEM((ZRF, L), jnp.float32),
            pltpu.SemaphoreType.DMA,
            pltpu.SemaphoreType.DMA,
            pltpu.SemaphoreType.DMA,
            pltpu.SemaphoreType.DMA,
        ],
    )
    def sc_pass(x_perm, vtx2d, edg2d, xv_out, acc1, xslab, xacc,
                vibuf, eibuf, rows, rows32, zb16, zb32,
                gsem, ssem, vsem, esem):
        c = lax.axis_index("core")
        w = lax.axis_index("sub")
        row_base = w * RPT

        @pl.loop(0, ZR)
        def _zero(i):
            zb16[i, :] = jnp.zeros((L,), DT)

        @pl.loop(0, ZRF)
        def _zerof(i):
            zb32[i, pl.ds(0, HL)] = jnp.zeros((HL,), jnp.float32)
            zb32[i, pl.ds(HL, HL)] = jnp.zeros((HL,), jnp.float32)

        def stream_blocks(gather_src, acc, gather_by_vertex, f32_acc):
            gibuf, gs = (vibuf, vsem) if gather_by_vertex else (eibuf, esem)
            sibuf, ss = (eibuf, esem) if gather_by_vertex else (vibuf, vsem)
            gsrc2d = vtx2d if gather_by_vertex else edg2d
            ssrc2d = edg2d if gather_by_vertex else vtx2d

            def load_idx(src2d, buf, sb, slot, sem):
                return pltpu.async_copy(
                    src2d.at[pl.ds(row_base + sb * K, K)], buf.at[slot], sem)

            def fire_g(slot):
                for k in range(K):
                    pltpu.async_copy(gather_src(gibuf.at[slot, k]),
                                     rows.at[slot, k], gsem)

            def drain_g(slot):
                for k in range(K):
                    pltpu.make_async_copy(gather_src(gibuf.at[slot, k]),
                                          rows.at[slot, k], gsem).wait()

            def fire_s(slot, f=0):
                for k in range(K):
                    src = rows32.at[f, k] if f32_acc else rows.at[slot, k]
                    pltpu.async_copy(src, acc.at[sibuf.at[slot, k]], ssem,
                                     add=True)

            def drain_s(slot, f=0):
                for k in range(K):
                    src = rows32.at[f, k] if f32_acc else rows.at[slot, k]
                    pltpu.make_async_copy(src, acc.at[sibuf.at[slot, k]],
                                          ssem).wait()

            def wait_idx(buf, slot, sem):
                pltpu.make_async_copy(gsrc2d.at[pl.ds(0, K)], buf.at[slot],
                                      sem).wait()

            def convert(slot, f):
                for k in range(K):
                    @pl.loop(0, BLK)
                    def _cv(i):
                        v = rows[slot, k, i, :]
                        a, b = plsc.unpack(
                            v, format=plsc.PackFormat.INTERLEAVED)
                        rows32[f, k, i, pl.ds(0, HL)] = a
                        rows32[f, k, i, pl.ds(HL, HL)] = b

            for b in (0, 1):
                load_idx(gsrc2d, gibuf, b, b, gs).wait()
                load_idx(ssrc2d, sibuf, b, b, ss).wait()
                fire_g(b)
            load_idx(gsrc2d, gibuf, 2, 2, gs)

            if not f32_acc:
                @pl.loop(0, NSB)
                def _sb(t):
                    b = lax.rem(t, RING)
                    drain_g(b)

                    @pl.when(t >= 2)
                    def _():
                        wait_idx(sibuf, b, ss)

                    fire_s(b)

                    @pl.when(t + 2 < NSB)
                    def _():
                        b2 = lax.rem(t + 2, RING)

                        @pl.when(t >= 2)
                        def _():
                            drain_s(b2)

                        load_idx(ssrc2d, sibuf, t + 2, b2, ss)

                        @pl.when(t + 3 < NSB)
                        def _():
                            load_idx(gsrc2d, gibuf, t + 3,
                                     lax.rem(t + 3, RING), gs)

                        wait_idx(gibuf, b2, gs)
                        fire_g(b2)

                for r in range(RING):
                    drain_s(r)
            else:
                @pl.loop(0, NSB)
                def _sb(t):
                    b = lax.rem(t, RING)
                    f = lax.rem(t, 2)
                    drain_g(b)

                    @pl.when(t >= 2)
                    def _():
                        drain_s(b, f)

                    convert(b, f)

                    @pl.when(t >= 2)
                    def _():
                        wait_idx(sibuf, b, ss)

                    fire_s(b, f)

                    @pl.when(t + 2 < NSB)
                    def _():
                        b2 = lax.rem(t + 2, RING)
                        load_idx(ssrc2d, sibuf, t + 2, b2, ss)

                        @pl.when(t + 3 < NSB)
                        def _():
                            load_idx(gsrc2d, gibuf, t + 3,
                                     lax.rem(t + 3, RING), gs)

                        wait_idx(gibuf, b2, gs)
                        fire_g(b2)

                drain_s(0, 0)
                drain_s(0, 1)

        for j in range(SPC):
            s = c * SPC + j
            cops = [pltpu.async_copy(
                x_perm.at[pl.ds(w * STRIPE_N, STRIPE_N), pl.ds(s * L, L)],
                xslab.at[pl.ds(w * STRIPE_N, STRIPE_N)], vsem)]
            for z in range(ZCOP):
                cops.append(pltpu.async_copy(
                    zb16.at[pl.ds(0, ZR)],
                    acc1.at[pl.ds(w * STRIPE_M + z * ZR, ZR)], vsem))
            for z in range(NZCOP):
                cops.append(pltpu.async_copy(
                    zb32.at[pl.ds(0, ZRF)],
                    xacc.at[pl.ds(w * STRIPE_N + z * ZRF, ZRF)], vsem))
            if NZREM:
                cops.append(pltpu.async_copy(
                    zb32.at[pl.ds(0, NZREM)],
                    xacc.at[pl.ds(w * STRIPE_N + NZCOP * ZRF, NZREM)], vsem))
            for cop in cops:
                cop.wait()
            plsc.subcore_barrier()
            stream_blocks(lambda i: xslab.at[i], acc1, True, False)
            plsc.subcore_barrier()
            stream_blocks(lambda i: acc1.at[i], xacc, False, True)
            plsc.subcore_barrier()
            pltpu.sync_copy(
                xacc.at[pl.ds(w * STRIPE_N, STRIPE_N)],
                xv_out.at[pl.ds(w * STRIPE_N, STRIPE_N), pl.ds(s * L, L)])

    return sc_pass


def _mlp_tc(xv, x0, w1, b1, w2, b2, *, interpret=False):
    N, D = xv.shape
    R = 1000 if N % 1000 == 0 else N
    grid = N // R

    def body(xv_ref, x0_ref, w1_ref, b1_ref, w2_ref, b2_ref, o_ref):
        xi = ((1.0 - ALPHA) * xv_ref[...].astype(jnp.float32)
              + ALPHA * x0_ref[...])
        h = jnp.maximum(
            jnp.dot(xi, w1_ref[...], preferred_element_type=jnp.float32)
            + b1_ref[...], 0.0)
        o = (jnp.dot(h, w2_ref[...], preferred_element_type=jnp.float32)
             + b2_ref[...])
        o_ref[...] = (1.0 - BETA) * xi + BETA * o

    return pl.pallas_call(
        body,
        grid=(grid,),
        in_specs=[
            pl.BlockSpec((R, D), lambda i: (i, 0)),
            pl.BlockSpec((R, D), lambda i: (i, 0)),
            pl.BlockSpec((D, D), lambda i: (0, 0)),
            pl.BlockSpec((1, D), lambda i: (0, 0)),
            pl.BlockSpec((D, D), lambda i: (0, 0)),
            pl.BlockSpec((1, D), lambda i: (0, 0)),
        ],
        out_specs=pl.BlockSpec((R, D), lambda i: (i, 0)),
        out_shape=jax.ShapeDtypeStruct((N, D), jnp.float32),
        interpret=interpret,
    )(xv, x0, w1, b1.reshape(1, D), w2, b2.reshape(1, D))


def _slice_perm(D, L):
    HL = L // 2
    perm = np.empty(D, np.int32)
    for s in range(D // L):
        for i in range(HL):
            perm[s * L + 2 * i] = s * L + i
            perm[s * L + 2 * i + 1] = s * L + HL + i
    return perm


def kernel(X, vertex, edges, X0, W1, b1, W2, b2):
    N, D = X.shape
    E = vertex.shape[0]
    M = M_EDGES
    BLK = 125
    vtx2d = vertex.reshape(E // BLK, BLK)
    edg2d = edges.reshape(E // BLK, BLK)
    x_perm = X.astype(jnp.bfloat16)[:, _slice_perm(D, 32)]
    sc = _sc_two_hop(N, E, M, D, BLK=BLK)
    xv = sc(x_perm, vtx2d, edg2d)
    return _mlp_tc(xv, X0, W1, b1, W2, b2)


# speedup vs baseline: 1.0624x; 1.0624x over previous
import functools

import jax
import jax.numpy as jnp
import numpy as np
from jax import lax
from jax.experimental import pallas as pl
from jax.experimental.pallas import tpu as pltpu
from jax.experimental.pallas import tpu_sc as plsc

ALPHA = 0.5
BETA = 1.0
M_EDGES = 80000


def _sc_two_hop(N, E, M, D, *, L=32, DT=jnp.bfloat16, NC=2, NS=16, BLK=125,
                K=1, RING=4, ZR=125, ZRF=50, interpret=False):
    HL = L // 2
    S = D // L
    SPC = S // NC
    SB = K * BLK
    NSB = E // (NS * SB)
    RPT = E // (NS * BLK)
    STRIPE_M = M // NS
    STRIPE_N = N // NS
    ZCOP = STRIPE_M // ZR
    NZCOP = STRIPE_N // ZRF
    NZREM = STRIPE_N - NZCOP * ZRF
    assert S * L == D and SPC * NC == S
    assert NS * SB * NSB == E and RPT == NSB * K
    assert ZCOP * ZR == STRIPE_M and STRIPE_N * NS == N
    assert BLK <= 128 and RING == 4 and NSB >= 4

    mesh = plsc.VectorSubcoreMesh(core_axis_name="core", subcore_axis_name="sub",
                                  num_cores=NC, num_subcores=NS)

    @functools.partial(
        pl.kernel,
        out_type=jax.ShapeDtypeStruct((N, D), jnp.float32),
        mesh=mesh,
        interpret=interpret,
        compiler_params=pltpu.CompilerParams(use_tc_tiling_on_sc=False,
                                             needs_layout_passes=False),
        scratch_types=[
            pltpu.VMEM_SHARED((M, L), DT),
            pltpu.VMEM_SHARED((N, L), DT),
            pltpu.VMEM_SHARED((N, L), jnp.float32),
            pltpu.VMEM((RING, K, BLK), jnp.int32),
            pltpu.VMEM((RING, K, BLK), jnp.int32),
            pltpu.VMEM((RING, K, BLK, L), DT),
            pltpu.VMEM((2, K, BLK, L), jnp.float32),
            pltpu.VMEM((ZR, L), DT),
            pltpu.VMEM((ZRF, L), jnp.float32),
            pltpu.SemaphoreType.DMA,
            pltpu.SemaphoreType.DMA,
            pltpu.SemaphoreType.DMA,
            pltpu.SemaphoreType.DMA,
        ],
    )
    def sc_pass(x_perm, vtx2d, edg2d, xv_out, acc1, xslab, xacc,
                vibuf, eibuf, rows, rows32, zb16, zb32,
                gsem, ssem, vsem, esem):
        c = lax.axis_index("core")
        w = lax.axis_index("sub")
        row_base = w * RPT

        @pl.loop(0, ZR)
        def _zero(i):
            zb16[i, :] = jnp.zeros((L,), DT)

        @pl.loop(0, ZRF)
        def _zerof(i):
            zb32[i, pl.ds(0, HL)] = jnp.zeros((HL,), jnp.float32)
            zb32[i, pl.ds(HL, HL)] = jnp.zeros((HL,), jnp.float32)

        def stream_blocks(gather_src, acc, gather_by_vertex, f32_acc):
            gibuf, gs = (vibuf, vsem) if gather_by_vertex else (eibuf, esem)
            sibuf, ss = (eibuf, esem) if gather_by_vertex else (vibuf, vsem)
            gsrc2d = vtx2d if gather_by_vertex else edg2d
            ssrc2d = edg2d if gather_by_vertex else vtx2d

            def load_idx(src2d, buf, sb, slot, sem):
                return pltpu.async_copy(
                    src2d.at[pl.ds(row_base + sb * K, K)], buf.at[slot], sem)

            def fire_g(slot):
                for k in range(K):
                    pltpu.async_copy(gather_src(gibuf.at[slot, k]),
                                     rows.at[slot, k], gsem)

            def drain_g(slot):
                for k in range(K):
                    pltpu.make_async_copy(gather_src(gibuf.at[slot, k]),
                                          rows.at[slot, k], gsem).wait()

            def fire_s(slot, f=0):
                for k in range(K):
                    src = rows32.at[f, k] if f32_acc else rows.at[slot, k]
                    pltpu.async_copy(src, acc.at[sibuf.at[slot, k]], ssem,
                                     add=True)

            def drain_s(slot, f=0):
                for k in range(K):
                    src = rows32.at[f, k] if f32_acc else rows.at[slot, k]
                    pltpu.make_async_copy(src, acc.at[sibuf.at[slot, k]],
                                          ssem).wait()

            def wait_idx(buf, slot, sem):
                pltpu.make_async_copy(gsrc2d.at[pl.ds(0, K)], buf.at[slot],
                                      sem).wait()

            def convert(slot, f):
                for k in range(K):
                    @pl.loop(0, BLK, unroll=5)
                    def _cv(i):
                        v = rows[slot, k, i, :]
                        a, b = plsc.unpack(
                            v, format=plsc.PackFormat.INTERLEAVED)
                        rows32[f, k, i, pl.ds(0, HL)] = a
                        rows32[f, k, i, pl.ds(HL, HL)] = b

            for b in (0, 1):
                load_idx(gsrc2d, gibuf, b, b, gs).wait()
                load_idx(ssrc2d, sibuf, b, b, ss).wait()
                fire_g(b)
            load_idx(gsrc2d, gibuf, 2, 2, gs)

            if not f32_acc:
                @pl.loop(0, NSB)
                def _sb(t):
                    b = lax.rem(t, RING)
                    drain_g(b)

                    @pl.when(t >= 2)
                    def _():
                        wait_idx(sibuf, b, ss)

                    fire_s(b)

                    @pl.when(t + 2 < NSB)
                    def _():
                        b2 = lax.rem(t + 2, RING)

                        @pl.when(t >= 2)
                        def _():
                            drain_s(b2)

                        load_idx(ssrc2d, sibuf, t + 2, b2, ss)

                        @pl.when(t + 3 < NSB)
                        def _():
                            load_idx(gsrc2d, gibuf, t + 3,
                                     lax.rem(t + 3, RING), gs)

                        wait_idx(gibuf, b2, gs)
                        fire_g(b2)

                for r in range(RING):
                    drain_s(r)
            else:
                @pl.loop(0, NSB)
                def _sb(t):
                    b = lax.rem(t, RING)
                    f = lax.rem(t, 2)
                    drain_g(b)

                    @pl.when(t >= 2)
                    def _():
                        drain_s(b, f)

                    convert(b, f)

                    @pl.when(t >= 2)
                    def _():
                        wait_idx(sibuf, b, ss)

                    fire_s(b, f)

                    @pl.when(t + 2 < NSB)
                    def _():
                        b2 = lax.rem(t + 2, RING)
                        load_idx(ssrc2d, sibuf, t + 2, b2, ss)

                        @pl.when(t + 3 < NSB)
                        def _():
                            load_idx(gsrc2d, gibuf, t + 3,
                                     lax.rem(t + 3, RING), gs)

                        wait_idx(gibuf, b2, gs)
                        fire_g(b2)

                drain_s(0, 0)
                drain_s(0, 1)

        for j in range(SPC):
            s = c * SPC + j
            cops = [pltpu.async_copy(
                x_perm.at[pl.ds(w * STRIPE_N, STRIPE_N), pl.ds(s * L, L)],
                xslab.at[pl.ds(w * STRIPE_N, STRIPE_N)], vsem)]
            for z in range(ZCOP):
                cops.append(pltpu.async_copy(
                    zb16.at[pl.ds(0, ZR)],
                    acc1.at[pl.ds(w * STRIPE_M + z * ZR, ZR)], vsem))
            for z in range(NZCOP):
                cops.append(pltpu.async_copy(
                    zb32.at[pl.ds(0, ZRF)],
                    xacc.at[pl.ds(w * STRIPE_N + z * ZRF, ZRF)], vsem))
            if NZREM:
                cops.append(pltpu.async_copy(
                    zb32.at[pl.ds(0, NZREM)],
                    xacc.at[pl.ds(w * STRIPE_N + NZCOP * ZRF, NZREM)], vsem))
            for cop in cops:
                cop.wait()
            plsc.subcore_barrier()
            stream_blocks(lambda i: xslab.at[i], acc1, True, False)
            plsc.subcore_barrier()
            stream_blocks(lambda i: acc1.at[i], xacc, False, True)
            plsc.subcore_barrier()
            pltpu.sync_copy(
                xacc.at[pl.ds(w * STRIPE_N, STRIPE_N)],
                xv_out.at[pl.ds(w * STRIPE_N, STRIPE_N), pl.ds(s * L, L)])

    return sc_pass


def _mlp_tc(xv, x0, w1, b1, w2, b2, *, interpret=False):
    N, D = xv.shape
    R = 1000 if N % 1000 == 0 else N
    grid = N // R

    def body(xv_ref, x0_ref, w1_ref, b1_ref, w2_ref, b2_ref, o_ref):
        xi = ((1.0 - ALPHA) * xv_ref[...].astype(jnp.float32)
              + ALPHA * x0_ref[...])
        h = jnp.maximum(
            jnp.dot(xi, w1_ref[...], preferred_element_type=jnp.float32)
            + b1_ref[...], 0.0)
        o = (jnp.dot(h, w2_ref[...], preferred_element_type=jnp.float32)
             + b2_ref[...])
        o_ref[...] = (1.0 - BETA) * xi + BETA * o

    return pl.pallas_call(
        body,
        grid=(grid,),
        in_specs=[
            pl.BlockSpec((R, D), lambda i: (i, 0)),
            pl.BlockSpec((R, D), lambda i: (i, 0)),
            pl.BlockSpec((D, D), lambda i: (0, 0)),
            pl.BlockSpec((1, D), lambda i: (0, 0)),
            pl.BlockSpec((D, D), lambda i: (0, 0)),
            pl.BlockSpec((1, D), lambda i: (0, 0)),
        ],
        out_specs=pl.BlockSpec((R, D), lambda i: (i, 0)),
        out_shape=jax.ShapeDtypeStruct((N, D), jnp.float32),
        interpret=interpret,
    )(xv, x0, w1, b1.reshape(1, D), w2, b2.reshape(1, D))


def _slice_perm(D, L):
    HL = L // 2
    perm = np.empty(D, np.int32)
    for s in range(D // L):
        for i in range(HL):
            perm[s * L + 2 * i] = s * L + i
            perm[s * L + 2 * i + 1] = s * L + HL + i
    return perm


def kernel(X, vertex, edges, X0, W1, b1, W2, b2):
    N, D = X.shape
    E = vertex.shape[0]
    M = M_EDGES
    BLK = 125
    vtx2d = vertex.reshape(E // BLK, BLK)
    edg2d = edges.reshape(E // BLK, BLK)
    x_perm = X.astype(jnp.bfloat16)[:, _slice_perm(D, 32)]
    sc = _sc_two_hop(N, E, M, D, BLK=BLK)
    xv = sc(x_perm, vtx2d, edg2d)
    return _mlp_tc(xv, X0, W1, b1, W2, b2)


# speedup vs baseline: 2.0531x; 1.9326x over previous
import functools

import jax
import jax.numpy as jnp
from jax import lax
from jax.experimental import pallas as pl
from jax.experimental.pallas import tpu as pltpu
from jax.experimental.pallas import tpu_sc as plsc

ALPHA = 0.5
BETA = 1.0
M_EDGES = 80000


def _sc_two_hop(N, E, M, D, *, L=32, DT=jnp.bfloat16, NC=2, NS=16, BLK=125,
                K=4, RING=4, ZR=125, interpret=False):
    S = D // L
    SPC = S // NC
    SB = K * BLK
    NSB = E // (NS * SB)
    RPT = E // (NS * BLK)
    STRIPE_M = M // NS
    STRIPE_N = N // NS
    ZCOP = STRIPE_M // ZR
    assert S * L == D and SPC * NC == S
    assert NS * SB * NSB == E and RPT == NSB * K
    assert ZCOP * ZR == STRIPE_M and STRIPE_N % ZR == 0 and STRIPE_N * NS == N
    assert BLK <= 128 and RING == 4 and NSB >= 4

    mesh = plsc.VectorSubcoreMesh(core_axis_name="core", subcore_axis_name="sub",
                                  num_cores=NC, num_subcores=NS)

    @functools.partial(
        pl.kernel,
        out_type=jax.ShapeDtypeStruct((N, D), DT),
        mesh=mesh,
        interpret=interpret,
        compiler_params=pltpu.CompilerParams(use_tc_tiling_on_sc=False),
        scratch_types=[
            pltpu.VMEM_SHARED((M, L), DT),
            pltpu.VMEM_SHARED((N, L), DT),
            pltpu.VMEM((RING, K, BLK), jnp.int32),
            pltpu.VMEM((RING, K, BLK), jnp.int32),
            pltpu.VMEM((RING, K, BLK, L), DT),
            pltpu.VMEM((ZR, L), DT),
            pltpu.SemaphoreType.DMA,
            pltpu.SemaphoreType.DMA,
            pltpu.SemaphoreType.DMA,
            pltpu.SemaphoreType.DMA,
        ],
    )
    def sc_pass(x_nat, vtx2d, edg2d, xv_out, acc1, xacc,
                vibuf, eibuf, rows, zbuf, gsem, ssem, vsem, esem):
        c = lax.axis_index("core")
        w = lax.axis_index("sub")
        row_base = w * RPT

        @pl.loop(0, ZR)
        def _zero(i):
            zbuf[i, :] = jnp.zeros((L,), DT)

        def stream_blocks(table, acc, gather_by_vertex):
            gibuf, gs = (vibuf, vsem) if gather_by_vertex else (eibuf, esem)
            sibuf, ss = (eibuf, esem) if gather_by_vertex else (vibuf, vsem)
            gsrc2d = vtx2d if gather_by_vertex else edg2d
            ssrc2d = edg2d if gather_by_vertex else vtx2d

            def load_idx(src2d, buf, sb, slot, sem):
                return pltpu.async_copy(
                    src2d.at[pl.ds(row_base + sb * K, K)], buf.at[slot], sem)

            def fire_g(slot):
                for k in range(K):
                    pltpu.async_copy(table.at[gibuf.at[slot, k]],
                                     rows.at[slot, k], gsem)

            def fire_s(slot):
                for k in range(K):
                    pltpu.async_copy(rows.at[slot, k],
                                     acc.at[sibuf.at[slot, k]], ssem, add=True)

            def drain_rows(sem, slot):
                for k in range(K):
                    pltpu.make_async_copy(table.at[gibuf.at[slot, k]],
                                          rows.at[slot, k], sem).wait()

            def wait_idx(buf, slot, sem):
                pltpu.make_async_copy(gsrc2d.at[pl.ds(0, K)], buf.at[slot],
                                      sem).wait()

            for b in (0, 1):
                load_idx(gsrc2d, gibuf, b, b, gs).wait()
                load_idx(ssrc2d, sibuf, b, b, ss).wait()
                fire_g(b)
            load_idx(gsrc2d, gibuf, 2, 2, gs)

            @pl.loop(0, NSB)
            def _sb(t):
                b = lax.rem(t, RING)
                drain_rows(gsem, b)

                @pl.when(t >= 2)
                def _():
                    wait_idx(sibuf, b, ss)

                fire_s(b)

                @pl.when(t + 2 < NSB)
                def _():
                    b2 = lax.rem(t + 2, RING)

                    @pl.when(t >= 2)
                    def _():
                        drain_rows(ssem, b2)

                    load_idx(ssrc2d, sibuf, t + 2, b2, ss)

                    @pl.when(t + 3 < NSB)
                    def _():
                        load_idx(gsrc2d, gibuf, t + 3, lax.rem(t + 3, RING),
                                 gs)

                    wait_idx(gibuf, b2, gs)
                    fire_g(b2)

            for r in range(RING):
                drain_rows(ssem, r)

        for j in range(SPC):
            s = c * SPC + j
            cops = [pltpu.async_copy(
                x_nat.at[pl.ds(w * STRIPE_N, STRIPE_N), pl.ds(s * L, L)],
                xacc.at[pl.ds(w * STRIPE_N, STRIPE_N)], vsem)]
            for z in range(ZCOP):
                cops.append(pltpu.async_copy(
                    zbuf.at[pl.ds(0, ZR)],
                    acc1.at[pl.ds(w * STRIPE_M + z * ZR, ZR)], vsem))
            for cop in cops:
                cop.wait()
            plsc.subcore_barrier()
            stream_blocks(xacc, acc1, True)
            plsc.subcore_barrier()
            for z in range(STRIPE_N // ZR):
                pltpu.sync_copy(zbuf.at[pl.ds(0, ZR)],
                                xacc.at[pl.ds(w * STRIPE_N + z * ZR, ZR)])
            plsc.subcore_barrier()
            stream_blocks(acc1, xacc, False)
            plsc.subcore_barrier()
            pltpu.sync_copy(
                xacc.at[pl.ds(w * STRIPE_N, STRIPE_N)],
                xv_out.at[pl.ds(w * STRIPE_N, STRIPE_N), pl.ds(s * L, L)])

    return sc_pass


def _mlp_tc(xv, x0, w1, b1, w2, b2, *, interpret=False):
    N, D = xv.shape
    R = 1000 if N % 1000 == 0 else N
    grid = N // R

    def body(xv_ref, x0_ref, w1_ref, b1_ref, w2_ref, b2_ref, o_ref):
        xi = ((1.0 - ALPHA) * xv_ref[...].astype(jnp.float32)
              + ALPHA * x0_ref[...])
        h = jnp.maximum(
            jnp.dot(xi, w1_ref[...], preferred_element_type=jnp.float32)
            + b1_ref[...], 0.0)
        o = (jnp.dot(h, w2_ref[...], preferred_element_type=jnp.float32)
             + b2_ref[...])
        o_ref[...] = (1.0 - BETA) * xi + BETA * o

    return pl.pallas_call(
        body,
        grid=(grid,),
        in_specs=[
            pl.BlockSpec((R, D), lambda i: (i, 0)),
            pl.BlockSpec((R, D), lambda i: (i, 0)),
            pl.BlockSpec((D, D), lambda i: (0, 0)),
            pl.BlockSpec((1, D), lambda i: (0, 0)),
            pl.BlockSpec((D, D), lambda i: (0, 0)),
            pl.BlockSpec((1, D), lambda i: (0, 0)),
        ],
        out_specs=pl.BlockSpec((R, D), lambda i: (i, 0)),
        out_shape=jax.ShapeDtypeStruct((N, D), jnp.float32),
        interpret=interpret,
    )(xv, x0, w1, b1.reshape(1, D), w2, b2.reshape(1, D))


def kernel(X, vertex, edges, X0, W1, b1, W2, b2):
    N, D = X.shape
    E = vertex.shape[0]
    M = M_EDGES
    BLK = 125
    vtx2d = vertex.reshape(E // BLK, BLK)
    edg2d = edges.reshape(E // BLK, BLK)
    sc = _sc_two_hop(N, E, M, D, BLK=BLK)
    xv = sc(X.astype(jnp.bfloat16), vtx2d, edg2d)
    return _mlp_tc(xv, X0, W1, b1, W2, b2)
